# baseline (speedup 1.0000x reference)
import functools

import jax
import jax.numpy as jnp
from jax import lax
from jax.experimental import pallas as pl
from jax.experimental.pallas import tpu as pltpu
from jax.experimental.pallas import tpu_sc as plsc

_B = 4096
_D = 64
_NW = 32
_LANES_W = _B // _NW * 8
_SUBROWS = 8


@functools.cache
def _build_sc_partial():
    mesh = plsc.VectorSubcoreMesh(core_axis_name="c", subcore_axis_name="s")

    @functools.partial(
        pl.kernel,
        mesh=mesh,
        out_type=jax.ShapeDtypeStruct((_NW, 16), jnp.float32),
        scratch_types=[
            pltpu.VMEM((_LANES_W,), jnp.float32),
            pltpu.VMEM((_SUBROWS, _LANES_W), jnp.float32),
            pltpu.VMEM((_SUBROWS, _LANES_W), jnp.float32),
            pltpu.VMEM((16,), jnp.float32),
            pltpu.SemaphoreType.DMA,
            pltpu.SemaphoreType.DMA,
        ],
        compiler_params=pltpu.CompilerParams(use_tc_tiling_on_sc=True),
    )
    def sc_partial(
        ids_hbm, yt_hbm, ypt_hbm, out_hbm, nv_v, y_v, yp_v, part_v, sem_y, sem_p
    ):
        wid = lax.axis_index("s") * 2 + lax.axis_index("c")
        jg = wid // 4
        q = wid % 4
        lbase = q * _LANES_W

        cp_y = pltpu.async_copy(
            yt_hbm.at[pl.ds(jg * _SUBROWS, _SUBROWS), pl.ds(lbase, _LANES_W)],
            y_v,
            sem_y,
        )
        cp_p = pltpu.async_copy(
            ypt_hbm.at[pl.ds(jg * _SUBROWS, _SUBROWS), pl.ds(lbase, _LANES_W)],
            yp_v,
            sem_p,
        )
        pltpu.sync_copy(ids_hbm.at[pl.ds(lbase, _LANES_W)], nv_v)

        def nv_body(c, _):
            i = nv_v[pl.ds(c * 16, 16)]
            nv_v[pl.ds(c * 16, 16)] = (
                16.0 + 8.0 * i - 4.0 * jnp.maximum(i - 5.0, 0.0)
            )
            return 0

        lax.fori_loop(0, _LANES_W // 16, nv_body, 0)
        cp_y.wait()
        cp_p.wait()

        def acc_body(c, acc):
            nv16 = nv_v[pl.ds(c * 16, 16)]
            for j in range(_SUBROWS):
                feat = jnp.full((16,), jg * _SUBROWS + j, dtype=jnp.float32)
                d = y_v[j, pl.ds(c * 16, 16)] - yp_v[j, pl.ds(c * 16, 16)]
                acc = acc + jnp.where(feat < nv16, d * d, 0.0)
            return acc

        acc = lax.fori_loop(
            0, _LANES_W // 16, acc_body, jnp.zeros((16,), jnp.float32)
        )
        part_v[...] = acc
        pltpu.sync_copy(part_v, out_hbm.at[wid])

    return sc_partial


def _reduce_body(p_ref, out_ref):
    out_ref[0, 0] = jnp.sum(p_ref[...]) * (1.0 / (_B * _D))


@jax.jit
def _masked_mse_sc(ids, yt, ypt):
    parts = _build_sc_partial()(ids, yt, ypt)
    out = pl.pallas_call(
        _reduce_body,
        out_specs=pl.BlockSpec(memory_space=pltpu.SMEM),
        out_shape=jax.ShapeDtypeStruct((1, 1), jnp.float32),
    )(parts)
    return out[0, 0]


def kernel(x, y, y_pred):
    ids = x[:, 0, 0]
    return _masked_mse_sc(ids, y.T, y_pred.T)


# speedup vs baseline: 1.0306x; 1.0306x over previous
import functools

import jax
import jax.numpy as jnp
from jax import lax
from jax.experimental import pallas as pl
from jax.experimental.pallas import tpu as pltpu
from jax.experimental.pallas import tpu_sc as plsc

_B = 4096
_D = 64
_NW = 32
_LANES_W = _B // _NW * 8
_SUBROWS = 8


@functools.cache
def _build_sc_partial():
    mesh = plsc.VectorSubcoreMesh(core_axis_name="c", subcore_axis_name="s")

    @functools.partial(
        pl.kernel,
        mesh=mesh,
        out_type=jax.ShapeDtypeStruct((_NW, 16), jnp.float32),
        scratch_types=[
            pltpu.VMEM((_LANES_W,), jnp.float32),
            pltpu.VMEM((_SUBROWS, _LANES_W), jnp.float32),
            pltpu.VMEM((_SUBROWS, _LANES_W), jnp.float32),
            pltpu.VMEM((16,), jnp.float32),
            pltpu.SemaphoreType.DMA,
            pltpu.SemaphoreType.DMA,
        ],
        compiler_params=pltpu.CompilerParams(use_tc_tiling_on_sc=True),
    )
    def sc_partial(
        ids_hbm, yt_hbm, ypt_hbm, out_hbm, nv_v, y_v, yp_v, part_v, sem_y, sem_p
    ):
        wid = lax.axis_index("s") * 2 + lax.axis_index("c")
        jg = wid // 4
        q = wid % 4
        lbase = q * _LANES_W

        cp_y = pltpu.async_copy(
            yt_hbm.at[pl.ds(jg * _SUBROWS, _SUBROWS), pl.ds(lbase, _LANES_W)],
            y_v,
            sem_y,
        )
        cp_p = pltpu.async_copy(
            ypt_hbm.at[pl.ds(jg * _SUBROWS, _SUBROWS), pl.ds(lbase, _LANES_W)],
            yp_v,
            sem_p,
        )
        pltpu.sync_copy(ids_hbm.at[pl.ds(lbase, _LANES_W)], nv_v)
        cp_y.wait()
        cp_p.wait()

        feats = tuple(
            jnp.full((16,), jg * _SUBROWS + j, dtype=jnp.float32)
            for j in range(_SUBROWS)
        )
        zero = jnp.zeros((16,), jnp.float32)

        @plsc.parallel_loop(0, _LANES_W, 16, unroll=4, carry=(zero, zero))
        def acc_loop(o, accs):
            acc0, acc1 = accs
            i = nv_v[pl.ds(o, 16)]
            nv16 = 16.0 + 8.0 * i - 4.0 * jnp.maximum(i - 5.0, 0.0)
            for j in range(_SUBROWS):
                d = y_v[j, pl.ds(o, 16)] - yp_v[j, pl.ds(o, 16)]
                t = jnp.where(feats[j] < nv16, d * d, 0.0)
                if j % 2 == 0:
                    acc0 = acc0 + t
                else:
                    acc1 = acc1 + t
            return acc0, acc1

        part_v[...] = acc_loop[0] + acc_loop[1]
        pltpu.sync_copy(part_v, out_hbm.at[wid])

    return sc_partial


def _reduce_body(p_ref, out_ref):
    out_ref[0, 0] = jnp.sum(p_ref[...]) * (1.0 / (_B * _D))


@jax.jit
def _masked_mse_sc(ids, yt, ypt):
    parts = _build_sc_partial()(ids, yt, ypt)
    out = pl.pallas_call(
        _reduce_body,
        out_specs=pl.BlockSpec(memory_space=pltpu.SMEM),
        out_shape=jax.ShapeDtypeStruct((1, 1), jnp.float32),
    )(parts)
    return out[0, 0]


def kernel(x, y, y_pred):
    ids = x[:, 0, 0]
    return _masked_mse_sc(ids, y.T, y_pred.T)


# speedup vs baseline: 1.0353x; 1.0046x over previous
import functools

import jax
import jax.numpy as jnp
from jax import lax
from jax.experimental import pallas as pl
from jax.experimental.pallas import tpu as pltpu
from jax.experimental.pallas import tpu_sc as plsc

_B = 4096
_D = 64
_NW = 32
_SUBROWS = 8

_TC_COLS = 3072
_SC_COLS = _B - _TC_COLS
_TC_BLK = 1024
_TC_GRID = _TC_COLS // _TC_BLK
_SC_CHUNK = _SC_COLS // (_NW // _SUBROWS)


@functools.cache
def _build_sc_partial():
    mesh = plsc.VectorSubcoreMesh(core_axis_name="c", subcore_axis_name="s")

    @functools.partial(
        pl.kernel,
        mesh=mesh,
        out_type=jax.ShapeDtypeStruct((_NW, 16), jnp.float32),
        scratch_types=[
            pltpu.VMEM((_SC_CHUNK,), jnp.float32),
            pltpu.VMEM((_SUBROWS, _SC_CHUNK), jnp.float32),
            pltpu.VMEM((_SUBROWS, _SC_CHUNK), jnp.float32),
            pltpu.VMEM((16,), jnp.float32),
            pltpu.SemaphoreType.DMA,
            pltpu.SemaphoreType.DMA,
        ],
        compiler_params=pltpu.CompilerParams(use_tc_tiling_on_sc=True),
    )
    def sc_partial(
        ids_hbm, yt_hbm, ypt_hbm, out_hbm, nv_v, y_v, yp_v, part_v, sem_y, sem_p
    ):
        wid = lax.axis_index("s") * 2 + lax.axis_index("c")
        jg = wid % _SUBROWS
        q = wid // _SUBROWS
        lbase = _TC_COLS + q * _SC_CHUNK

        cp_y = pltpu.async_copy(
            yt_hbm.at[pl.ds(jg * _SUBROWS, _SUBROWS), pl.ds(lbase, _SC_CHUNK)],
            y_v,
            sem_y,
        )
        cp_p = pltpu.async_copy(
            ypt_hbm.at[pl.ds(jg * _SUBROWS, _SUBROWS), pl.ds(lbase, _SC_CHUNK)],
            yp_v,
            sem_p,
        )
        pltpu.sync_copy(ids_hbm.at[pl.ds(lbase, _SC_CHUNK)], nv_v)

        def nv_body(c, _):
            i = nv_v[pl.ds(c * 16, 16)]
            nv_v[pl.ds(c * 16, 16)] = (
                16.0 + 8.0 * i - 4.0 * jnp.maximum(i - 5.0, 0.0)
            )
            return 0

        lax.fori_loop(0, _SC_CHUNK // 16, nv_body, 0)
        cp_y.wait()
        cp_p.wait()

        def acc_body(c, acc):
            nv16 = nv_v[pl.ds(c * 16, 16)]
            for j in range(_SUBROWS):
                feat = jnp.full((16,), jg * _SUBROWS + j, dtype=jnp.float32)
                d = y_v[j, pl.ds(c * 16, 16)] - yp_v[j, pl.ds(c * 16, 16)]
                acc = acc + jnp.where(feat < nv16, d * d, 0.0)
            return acc

        acc = lax.fori_loop(
            0, _SC_CHUNK // 16, acc_body, jnp.zeros((16,), jnp.float32)
        )
        part_v[...] = acc
        pltpu.sync_copy(part_v, out_hbm.at[wid])

    return sc_partial


def _tc_body(ids_ref, y_ref, yp_ref, out_ref):
    step = pl.program_id(0)

    ids = ids_ref[...].astype(jnp.int32)
    nv = 16 + 8 * ids - 4 * jnp.maximum(ids - 5, 0)
    nvb = jnp.broadcast_to(nv, (_D, _TC_BLK))
    feat = lax.broadcasted_iota(jnp.int32, (_D, _TC_BLK), 0)
    diff = y_ref[...] - yp_ref[...]
    part = jnp.sum(jnp.where(feat < nvb, diff * diff, 0.0))

    @pl.when(step == 0)
    def _():
        out_ref[0, 0] = 0.0

    out_ref[0, 0] += part


def _combine_body(tc_ref, sc_ref, out_ref):
    out_ref[0, 0] = (tc_ref[0, 0] + jnp.sum(sc_ref[...])) * (
        1.0 / (_B * _D)
    )


@jax.jit
def _masked_mse_hybrid(ids_f, ids1d, yt, ypt):
    sc_parts = _build_sc_partial()(ids1d, yt, ypt)
    tc_part = pl.pallas_call(
        _tc_body,
        grid=(_TC_GRID,),
        in_specs=[
            pl.BlockSpec((1, _TC_BLK), lambda i: (0, i)),
            pl.BlockSpec((_D, _TC_BLK), lambda i: (0, i)),
            pl.BlockSpec((_D, _TC_BLK), lambda i: (0, i)),
        ],
        out_specs=pl.BlockSpec(
            (1, 1), lambda i: (0, 0), memory_space=pltpu.SMEM
        ),
        out_shape=jax.ShapeDtypeStruct((1, 1), jnp.float32),
        compiler_params=pltpu.CompilerParams(
            allow_input_fusion=[True, False, False],
        ),
    )(ids_f, yt, ypt)
    out = pl.pallas_call(
        _combine_body,
        in_specs=[
            pl.BlockSpec(memory_space=pltpu.SMEM),
            pl.BlockSpec(memory_space=pltpu.VMEM),
        ],
        out_specs=pl.BlockSpec(memory_space=pltpu.SMEM),
        out_shape=jax.ShapeDtypeStruct((1, 1), jnp.float32),
    )(tc_part, sc_parts)
    return out[0, 0]


def kernel(x, y, y_pred):
    ids_f = x[:, 0, 0].reshape(1, _B)
    ids1d = x[:, 0, 0]
    return _masked_mse_hybrid(ids_f, ids1d, y.T, y_pred.T)


# speedup vs baseline: 8.7717x; 8.4728x over previous
import jax
import jax.numpy as jnp
from jax import lax
from jax.experimental import pallas as pl
from jax.experimental.pallas import tpu as pltpu

_OUT_DIM = 64
_B = 4096
_COLS_PER_BLK = 2048
_GRID = _B // _COLS_PER_BLK


def _mse_body(ids_ref, y_ref, yp_ref, out_ref):
    step = pl.program_id(0)

    ids = ids_ref[...].astype(jnp.int32)
    nv = 16 + 8 * ids - 4 * jnp.maximum(ids - 5, 0)
    nvb = jnp.broadcast_to(nv, (_OUT_DIM, _COLS_PER_BLK))
    feat = lax.broadcasted_iota(jnp.int32, (_OUT_DIM, _COLS_PER_BLK), 0)
    diff = y_ref[...] - yp_ref[...]
    part = jnp.sum(jnp.where(feat < nvb, diff * diff, 0.0))

    @pl.when(step == 0)
    def _():
        out_ref[0, 0] = 0.0

    out_ref[0, 0] += part

    @pl.when(step == _GRID - 1)
    def _():
        out_ref[0, 0] = out_ref[0, 0] * (1.0 / (_B * _OUT_DIM))


@jax.jit
def _masked_mse(ids_f, yt, ypt):
    out = pl.pallas_call(
        _mse_body,
        grid=(_GRID,),
        in_specs=[
            pl.BlockSpec((1, _COLS_PER_BLK), lambda i: (0, i)),
            pl.BlockSpec((_OUT_DIM, _COLS_PER_BLK), lambda i: (0, i)),
            pl.BlockSpec((_OUT_DIM, _COLS_PER_BLK), lambda i: (0, i)),
        ],
        out_specs=pl.BlockSpec(
            (1, 1), lambda i: (0, 0), memory_space=pltpu.SMEM
        ),
        out_shape=jax.ShapeDtypeStruct((1, 1), jnp.float32),
        compiler_params=pltpu.CompilerParams(
            allow_input_fusion=[True, False, False],
        ),
    )(ids_f, yt, ypt)
    return out[0, 0]


def kernel(x, y, y_pred):
    ids_f = x[:, 0, 0].reshape(1, _B)
    return _masked_mse(ids_f, y.T, y_pred.T)
